# Initial kernel scaffold; baseline (speedup 1.0000x reference)
import jax
import jax.numpy as jnp
from jax.experimental import pallas as pl


def kernel(x, edge_index, edge_attr, t, s, W_emb, b_emb, W_st, b_st, ln_g, ln_b, Wq, bq, Wk, bk, Wv, bv, We, Wskip, bskip, W_dec, b_dec):
    raise NotImplementedError("write your pallas kernel here")


# speedup vs baseline: 1.6369x; 1.6369x over previous
import functools

import jax
import jax.numpy as jnp
from jax import lax
from jax.experimental import pallas as pl
from jax.experimental.pallas import tpu as pltpu
from jax.experimental.pallas import tpu_sc as plsc

N = 10000
E = 160000
HID = 256
EDIM = 16
L = 3
FH = 6
SCALE = 1.0 / 16.0

NC = 2
NS = 16
NW = NC * NS

E_PAD = 163840
EPT_A = E_PAD // NW
EPT_B = E_PAD // NS
CA = 64
CB = 64
HALF = 128
ROWS_PER_TILE = N // NS
ZR = 25

BN = 1000


def _prologue_body(x_ref, t_ref, s_ref, wemb_ref, bemb_ref, wsth_ref,
                   wstt_ref, wsts_ref, bst_ref, lng_ref, lnb_ref, o_ref):
    h1 = jnp.dot(x_ref[...], wemb_ref[...],
                 preferred_element_type=jnp.float32) + bemb_ref[...]
    h2 = (jnp.dot(h1, wsth_ref[...], preferred_element_type=jnp.float32)
          + jnp.dot(t_ref[...], wstt_ref[...], preferred_element_type=jnp.float32)
          + jnp.dot(s_ref[...], wsts_ref[...], preferred_element_type=jnp.float32)
          + bst_ref[...])
    mu = jnp.mean(h2, axis=1, keepdims=True)
    var = jnp.mean((h2 - mu) ** 2, axis=1, keepdims=True)
    h = (h2 - mu) * lax.rsqrt(var + 1e-5) * lng_ref[...] + lnb_ref[...]
    o_ref[...] = jnp.maximum(h, 0.0)


def _stageq_body(h_ref, wq_ref, bq_ref, wk_ref, bk_ref, wv_ref, bv_ref,
                 wet_ref, q_ref, k_ref, qw_ref, v0_ref, v1_ref):
    h = h_ref[...]
    q = jnp.dot(h, wq_ref[...], preferred_element_type=jnp.float32) + bq_ref[...]
    k = jnp.dot(h, wk_ref[...], preferred_element_type=jnp.float32) + bk_ref[...]
    v = jnp.dot(h, wv_ref[...], preferred_element_type=jnp.float32) + bv_ref[...]
    q_ref[...] = q
    k_ref[...] = k
    qw_ref[...] = jnp.dot(q, wet_ref[...], preferred_element_type=jnp.float32)
    v0_ref[...] = v[:, :HALF]
    v1_ref[...] = v[:, HALF:]


def _epilogue_body(h_ref, u0_ref, u1_ref, t_ref, we_ref, wskip_ref, bskip_ref,
                   o_ref):
    h = h_ref[...]
    tt = t_ref[...]
    a16 = tt[:, :EDIM]
    den = jnp.maximum(tt[:, EDIM:EDIM + 1], 1e-16)
    u = jnp.concatenate([u0_ref[...], u1_ref[...]], axis=1)
    agg = (u + jnp.dot(a16, we_ref[...], preferred_element_type=jnp.float32)) / den
    out = agg + jnp.dot(h, wskip_ref[...],
                        preferred_element_type=jnp.float32) + bskip_ref[...]
    o_ref[...] = h + jnp.maximum(out, 0.0)


def _decoder_body(h_ref, wd_ref, bd_ref, o_ref):
    o_ref[...] = jnp.dot(h_ref[...], wd_ref[...],
                         preferred_element_type=jnp.float32) + bd_ref[...]


def _row_spec(cols):
    return pl.BlockSpec((BN, cols), lambda i: (i, 0))


def _full_spec(rows, cols):
    return pl.BlockSpec((rows, cols), lambda i: (0, 0))


def _tc_prologue(x96, t, s, W_emb, b_emb, Wst_h, Wst_t, Wst_s, b_st, ln_g, ln_b):
    return pl.pallas_call(
        _prologue_body,
        grid=(N // BN,),
        in_specs=[
            _row_spec(96), _row_spec(4), _row_spec(6),
            _full_spec(96, HID), _full_spec(1, HID),
            _full_spec(HID, HID), _full_spec(4, HID), _full_spec(6, HID),
            _full_spec(1, HID), _full_spec(1, HID), _full_spec(1, HID),
        ],
        out_specs=_row_spec(HID),
        out_shape=jax.ShapeDtypeStruct((N, HID), jnp.float32),
    )(x96, t, s, W_emb, b_emb, Wst_h, Wst_t, Wst_s, b_st, ln_g, ln_b)


def _tc_stageq(h, Wq, bq, Wk, bk, Wv, bv, WeT):
    return pl.pallas_call(
        _stageq_body,
        grid=(N // BN,),
        in_specs=[
            _row_spec(HID),
            _full_spec(HID, HID), _full_spec(1, HID),
            _full_spec(HID, HID), _full_spec(1, HID),
            _full_spec(HID, HID), _full_spec(1, HID),
            _full_spec(HID, EDIM),
        ],
        out_specs=[
            _row_spec(HID), _row_spec(HID), _row_spec(EDIM),
            _row_spec(HALF), _row_spec(HALF),
        ],
        out_shape=[
            jax.ShapeDtypeStruct((N, HID), jnp.float32),
            jax.ShapeDtypeStruct((N, HID), jnp.float32),
            jax.ShapeDtypeStruct((N, EDIM), jnp.float32),
            jax.ShapeDtypeStruct((N, HALF), jnp.float32),
            jax.ShapeDtypeStruct((N, HALF), jnp.float32),
        ],
    )(h, Wq, bq, Wk, bk, Wv, bv, WeT)


def _tc_epilogue(h, U0, U1, T, We, Wskip, bskip):
    return pl.pallas_call(
        _epilogue_body,
        grid=(N // BN,),
        in_specs=[
            _row_spec(HID), _row_spec(HALF), _row_spec(HALF), _row_spec(32),
            _full_spec(EDIM, HID), _full_spec(HID, HID), _full_spec(1, HID),
        ],
        out_specs=_row_spec(HID),
        out_shape=jax.ShapeDtypeStruct((N, HID), jnp.float32),
    )(h, U0, U1, T, We, Wskip, bskip)


def _tc_decoder(h, W_dec, b_dec):
    return pl.pallas_call(
        _decoder_body,
        grid=(N // BN,),
        in_specs=[
            _row_spec(HID), _full_spec(HID, 48), _full_spec(1, 48),
        ],
        out_specs=_row_spec(48),
        out_shape=jax.ShapeDtypeStruct((N, 48), jnp.float32),
    )(h, W_dec, b_dec)


_MESH = plsc.VectorSubcoreMesh(core_axis_name="c", subcore_axis_name="s",
                               num_cores=NC, num_subcores=NS)


def _passa_body(q_hbm, k_hbm, qw_hbm, src_hbm, dst_hbm, attr_hbm, ea_hbm,
                sidx, didx, qbuf, kbuf, qwbuf, attrbuf, eabuf, s1, s2, s3):
    cid = lax.axis_index("c")
    sid = lax.axis_index("s")
    wid = sid * NC + cid
    base = wid * EPT_A
    lane = lax.iota(jnp.int32, 16)

    def chunk(ci, carry):
        off = base + ci * CA
        pltpu.sync_copy(src_hbm.at[pl.ds(off, CA)], sidx)
        pltpu.sync_copy(dst_hbm.at[pl.ds(off, CA)], didx)
        cq = pltpu.async_copy(q_hbm.at[didx], qbuf, s1)
        ck = pltpu.async_copy(k_hbm.at[sidx], kbuf, s2)
        cw = pltpu.async_copy(qw_hbm.at[didx], qwbuf, s3)
        pltpu.sync_copy(attr_hbm.at[pl.ds(off, CA), :], attrbuf)
        cq.wait()
        ck.wait()
        cw.wait()

        def group(g, carry2):
            rows = g * 16 + lane

            def fblock(fb, acc):
                for j in range(16):
                    fvec = jnp.full((16,), fb * 16 + j, jnp.int32)
                    qv = plsc.load_gather(qbuf, [rows, fvec])
                    kv = plsc.load_gather(kbuf, [rows, fvec])
                    acc = acc + qv * kv
                return acc

            avec = lax.fori_loop(0, 16, fblock, jnp.zeros((16,), jnp.float32))
            for j in range(EDIM):
                fvec = jnp.full((16,), j, jnp.int32)
                av = plsc.load_gather(attrbuf, [rows, fvec])
                wv = plsc.load_gather(qwbuf, [rows, fvec])
                avec = avec + av * wv
            eav = jnp.exp(avec * SCALE)
            gid = off + g * 16 + lane
            eav = jnp.where(gid < E, eav, 0.0)
            eabuf[pl.ds(ci * CA + g * 16, 16)] = eav
            return carry2

        lax.fori_loop(0, CA // 16, group, 0)
        return carry

    lax.fori_loop(0, EPT_A // CA, chunk, 0)
    pltpu.sync_copy(eabuf, ea_hbm.at[pl.ds(base, EPT_A)])


_passa = functools.partial(
    pl.kernel,
    out_type=jax.ShapeDtypeStruct((E_PAD,), jnp.float32),
    mesh=_MESH,
    compiler_params=pltpu.CompilerParams(use_tc_tiling_on_sc=False, needs_layout_passes=False),
    scratch_types=[
        pltpu.VMEM((CA,), jnp.int32),
        pltpu.VMEM((CA,), jnp.int32),
        pltpu.VMEM((CA, HID), jnp.float32),
        pltpu.VMEM((CA, HID), jnp.float32),
        pltpu.VMEM((CA, EDIM), jnp.float32),
        pltpu.VMEM((CA, EDIM), jnp.float32),
        pltpu.VMEM((EPT_A,), jnp.float32),
        pltpu.SemaphoreType.DMA,
        pltpu.SemaphoreType.DMA,
        pltpu.SemaphoreType.DMA,
    ],
)(_passa_body)


def _passb_body(v0_hbm, v1_hbm, src_hbm, dst_hbm, attr_hbm, ea_hbm,
                u0_hbm, u1_hbm, t_hbm,
                sidx, didx, vbuf, trow, eab, attrb, zbuf, tzbuf, u_sp, t_sp,
                s1):
    cid = lax.axis_index("c")
    sid = lax.axis_index("s")
    lane = lax.iota(jnp.int32, 16)
    zero16 = jnp.zeros((16,), jnp.float32)

    def zrow(r, c):
        for fb in range(HALF // 16):
            zbuf[r, pl.ds(fb * 16, 16)] = zero16
        tzbuf[r, pl.ds(0, 16)] = zero16
        tzbuf[r, pl.ds(16, 16)] = zero16
        return c

    lax.fori_loop(0, ZR, zrow, 0)
    rbase = sid * ROWS_PER_TILE

    def zinit(i, c):
        pltpu.sync_copy(zbuf, u_sp.at[pl.ds(rbase + i * ZR, ZR)])
        pltpu.sync_copy(tzbuf, t_sp.at[pl.ds(rbase + i * ZR, ZR)])
        return c

    lax.fori_loop(0, ROWS_PER_TILE // ZR, zinit, 0)
    plsc.subcore_barrier()

    base = sid * EPT_B

    def chunk(ci, carry):
        off = base + ci * CB
        pltpu.sync_copy(src_hbm.at[pl.ds(off, CB)], sidx)
        pltpu.sync_copy(dst_hbm.at[pl.ds(off, CB)], didx)
        pltpu.sync_copy(ea_hbm.at[pl.ds(off, CB)], eab)
        pltpu.sync_copy(attr_hbm.at[pl.ds(off, CB), :], attrb)

        @pl.when(cid == 0)
        def _():
            pltpu.async_copy(v0_hbm.at[sidx], vbuf, s1).wait()

        @pl.when(cid == 1)
        def _():
            pltpu.async_copy(v1_hbm.at[sidx], vbuf, s1).wait()

        def edge(e, carry2):
            ea_b = plsc.load_gather(eab, [jnp.full((16,), e, jnp.int32)])
            for fb in range(HALF // 16):
                vbuf[e, pl.ds(fb * 16, 16)] = vbuf[e, pl.ds(fb * 16, 16)] * ea_b
            trow[e, pl.ds(0, 16)] = attrb[e, :] * ea_b
            trow[e, pl.ds(16, 16)] = jnp.where(lane == 0, ea_b, 0.0)
            return carry2

        lax.fori_loop(0, CB, edge, 0)
        pltpu.sync_copy(vbuf, u_sp.at[didx], add=True)

        @pl.when(cid == 1)
        def _():
            pltpu.sync_copy(trow, t_sp.at[didx], add=True)

        return carry

    lax.fori_loop(0, EPT_B // CB, chunk, 0)
    plsc.subcore_barrier()

    @pl.when(cid == 0)
    def _():
        pltpu.sync_copy(u_sp.at[pl.ds(rbase, ROWS_PER_TILE)],
                        u0_hbm.at[pl.ds(rbase, ROWS_PER_TILE)])

    @pl.when(cid == 1)
    def _():
        pltpu.sync_copy(u_sp.at[pl.ds(rbase, ROWS_PER_TILE)],
                        u1_hbm.at[pl.ds(rbase, ROWS_PER_TILE)])
        pltpu.sync_copy(t_sp.at[pl.ds(rbase, ROWS_PER_TILE)],
                        t_hbm.at[pl.ds(rbase, ROWS_PER_TILE)])


_passb = functools.partial(
    pl.kernel,
    out_type=[
        jax.ShapeDtypeStruct((N, HALF), jnp.float32),
        jax.ShapeDtypeStruct((N, HALF), jnp.float32),
        jax.ShapeDtypeStruct((N, 32), jnp.float32),
    ],
    mesh=_MESH,
    compiler_params=pltpu.CompilerParams(use_tc_tiling_on_sc=False, needs_layout_passes=False),
    scratch_types=[
        pltpu.VMEM((CB,), jnp.int32),
        pltpu.VMEM((CB,), jnp.int32),
        pltpu.VMEM((CB, HALF), jnp.float32),
        pltpu.VMEM((CB, 32), jnp.float32),
        pltpu.VMEM((CB,), jnp.float32),
        pltpu.VMEM((CB, EDIM), jnp.float32),
        pltpu.VMEM((ZR, HALF), jnp.float32),
        pltpu.VMEM((ZR, 32), jnp.float32),
        pltpu.VMEM_SHARED((N, HALF), jnp.float32),
        pltpu.VMEM_SHARED((N, 32), jnp.float32),
        pltpu.SemaphoreType.DMA,
    ],
)(_passb_body)


@jax.jit
def kernel(x, edge_index, edge_attr, t, s, W_emb, b_emb, W_st, b_st, ln_g,
           ln_b, Wq, bq, Wk, bk, Wv, bv, We, Wskip, bskip, W_dec, b_dec):
    x96 = x.reshape(N, -1)
    r2 = lambda a: a.reshape(1, -1)

    h = _tc_prologue(x96, t, s, W_emb, r2(b_emb), W_st[:HID], W_st[HID:HID + 4],
                     W_st[HID + 4:], r2(b_st), r2(ln_g), r2(ln_b))

    pad = E_PAD - E
    srcp = jnp.concatenate([edge_index[0], jnp.zeros((pad,), jnp.int32)])
    dstp = jnp.concatenate([edge_index[1], jnp.zeros((pad,), jnp.int32)])
    attrp = jnp.concatenate([edge_attr, jnp.zeros((pad, EDIM), jnp.float32)])

    for l in range(L):
        q, k, qw, v0, v1 = _tc_stageq(h, Wq[l], r2(bq[l]), Wk[l], r2(bk[l]),
                                      Wv[l], r2(bv[l]), We[l].T)
        ea = _passa(q, k, qw, srcp, dstp, attrp)
        U0, U1, T = _passb(v0, v1, srcp, dstp, attrp, ea)
        h = _tc_epilogue(h, U0, U1, T, We[l], Wskip[l], r2(bskip[l]))

    o = _tc_decoder(h, W_dec, r2(b_dec))
    return o.reshape(N, -1, FH)
